# baseline (speedup 1.0000x reference)
import jax
import jax.numpy as jnp
from jax.experimental import pallas as pl
from jax.experimental.pallas import tpu as pltpu

HIDDEN_DIM = 1024
NUM_BLOCKS = 32
BLOCK_SIZE = 16
NUM_CLUSTERS = 4
HIDDEN_SIZE = 128

N_TILE = 8


def _fused_kernel(x_ref, w1_ref, b1_ref, w2_ref, b2_ref, c_ref, out_ref,
                  gate_scr):
    @pl.when(pl.program_id(1) == 0)
    def _compute_gate():
        x = x_ref[0]
        h = x @ w1_ref[...] + b1_ref[...]
        h = 0.5 * h * (1.0 + jax.lax.erf(h * 0.7071067811865476))
        g = jax.lax.dot_general(
            w2_ref[...], h, (((0,), (1,)), ((), ())),
            preferred_element_type=jnp.float32) + b2_ref[...]
        g = g - jnp.max(g, axis=0, keepdims=True)
        e = jnp.exp(g)
        gate_scr[...] = e / jnp.sum(e, axis=0, keepdims=True)

    acc = jax.lax.dot_general(
        c_ref[...], gate_scr[...], (((1,), (0,)), ((), ())),
        preferred_element_type=jnp.float32)
    out_ref[...] = acc.reshape(1, N_TILE, BLOCK_SIZE, BLOCK_SIZE,
                               acc.shape[-1])


def kernel(tensor, W1, b1, W2, b2, cluster_logits):
    B, SEQ, _ = tensor.shape
    b1r = b1.reshape(1, HIDDEN_SIZE)
    b2r = b2.reshape(NUM_CLUSTERS, 1)
    cT = cluster_logits.reshape(NUM_CLUSTERS, -1).T

    grid = (B, NUM_BLOCKS // N_TILE)
    out = pl.pallas_call(
        _fused_kernel,
        grid=grid,
        in_specs=[
            pl.BlockSpec((1, SEQ, HIDDEN_DIM), lambda b, n: (b, 0, 0)),
            pl.BlockSpec((HIDDEN_DIM, HIDDEN_SIZE), lambda b, n: (0, 0)),
            pl.BlockSpec((1, HIDDEN_SIZE), lambda b, n: (0, 0)),
            pl.BlockSpec((HIDDEN_SIZE, NUM_CLUSTERS), lambda b, n: (0, 0)),
            pl.BlockSpec((NUM_CLUSTERS, 1), lambda b, n: (0, 0)),
            pl.BlockSpec((N_TILE * BLOCK_SIZE * BLOCK_SIZE, NUM_CLUSTERS),
                         lambda b, n: (n, 0)),
        ],
        out_specs=pl.BlockSpec(
            (1, N_TILE, BLOCK_SIZE, BLOCK_SIZE, SEQ),
            lambda b, n: (b, n, 0, 0, 0)),
        out_shape=jax.ShapeDtypeStruct(
            (B, NUM_BLOCKS, BLOCK_SIZE, BLOCK_SIZE, SEQ), jnp.float32),
        scratch_shapes=[pltpu.VMEM((NUM_CLUSTERS, SEQ), jnp.float32)],
    )(tensor, W1, b1r, W2, b2r, cT)
    return jnp.transpose(out, (0, 4, 1, 2, 3))


# speedup vs baseline: 1.0103x; 1.0103x over previous
import jax
import jax.numpy as jnp
from jax.experimental import pallas as pl
from jax.experimental.pallas import tpu as pltpu

HIDDEN_DIM = 1024
NUM_BLOCKS = 32
BLOCK_SIZE = 16
NUM_CLUSTERS = 4
HIDDEN_SIZE = 128

N_TILE = 8
NS = NUM_BLOCKS // N_TILE


def _fused_kernel(x_ref, w1_ref, b1_ref, w2_ref, b2_ref, c_ref, out_ref,
                  gate_scr):
    t = pl.program_id(0)

    @pl.when(jnp.logical_or(t == 0, t == NS))
    def _compute_gate():
        x = x_ref[0]
        h = x @ w1_ref[...] + b1_ref[...]
        h = 0.5 * h * (1.0 + jax.lax.erf(h * 0.7071067811865476))
        g = jax.lax.dot_general(
            w2_ref[...], h, (((0,), (1,)), ((), ())),
            preferred_element_type=jnp.float32) + b2_ref[...]
        g = g - jnp.max(g, axis=0, keepdims=True)
        e = jnp.exp(g)
        gate_scr[t // NS] = e / jnp.sum(e, axis=0, keepdims=True)

    @pl.when(t > 0)
    def _expand():
        q = jnp.minimum((t - 1) // NS, 1)
        acc = jax.lax.dot_general(
            c_ref[...], gate_scr[q], (((1,), (0,)), ((), ())),
            preferred_element_type=jnp.float32)
        out_ref[...] = acc.reshape(1, N_TILE, BLOCK_SIZE, BLOCK_SIZE,
                                   acc.shape[-1])


def kernel(tensor, W1, b1, W2, b2, cluster_logits):
    B, SEQ, _ = tensor.shape
    b1r = b1.reshape(1, HIDDEN_SIZE)
    b2r = b2.reshape(NUM_CLUSTERS, 1)
    cT = cluster_logits.reshape(NUM_CLUSTERS, -1).T

    def bo(t):
        return jnp.minimum(jnp.maximum(t - 1, 0) // NS, B - 1)

    def no(t):
        return jnp.maximum(t - 1, 0) % NS

    grid = (1 + B * NS,)
    out = pl.pallas_call(
        _fused_kernel,
        grid=grid,
        in_specs=[
            pl.BlockSpec((1, SEQ, HIDDEN_DIM),
                         lambda t: (jnp.minimum(t // NS, B - 1), 0, 0)),
            pl.BlockSpec((HIDDEN_DIM, HIDDEN_SIZE), lambda t: (0, 0)),
            pl.BlockSpec((1, HIDDEN_SIZE), lambda t: (0, 0)),
            pl.BlockSpec((HIDDEN_SIZE, NUM_CLUSTERS), lambda t: (0, 0)),
            pl.BlockSpec((NUM_CLUSTERS, 1), lambda t: (0, 0)),
            pl.BlockSpec((N_TILE * BLOCK_SIZE * BLOCK_SIZE, NUM_CLUSTERS),
                         lambda t: (no(t), 0)),
        ],
        out_specs=pl.BlockSpec(
            (1, N_TILE, BLOCK_SIZE, BLOCK_SIZE, SEQ),
            lambda t: (bo(t), no(t), 0, 0, 0)),
        out_shape=jax.ShapeDtypeStruct(
            (B, NUM_BLOCKS, BLOCK_SIZE, BLOCK_SIZE, SEQ), jnp.float32),
        scratch_shapes=[pltpu.VMEM((B, NUM_CLUSTERS, SEQ), jnp.float32)],
    )(tensor, W1, b1r, W2, b2r, cT)
    return jnp.transpose(out, (0, 4, 1, 2, 3))


# speedup vs baseline: 1.0635x; 1.0527x over previous
import jax
import jax.numpy as jnp
from jax.experimental import pallas as pl

HIDDEN_DIM = 1024
NUM_BLOCKS = 32
BLOCK_SIZE = 16
NUM_CLUSTERS = 4
HIDDEN_SIZE = 128

S_TILE = 512


def _fused_kernel(x_ref, w1_ref, b1_ref, w2_ref, b2_ref, c_ref, out_ref):
    x = x_ref[0]
    h = x @ w1_ref[...] + b1_ref[...]
    h = 0.5 * h * (1.0 + jax.lax.erf(h * 0.7071067811865476))
    g = jax.lax.dot_general(
        w2_ref[...], h, (((0,), (1,)), ((), ())),
        preferred_element_type=jnp.float32) + b2_ref[...]
    g = g - jnp.max(g, axis=0, keepdims=True)
    e = jnp.exp(g)
    gate = e / jnp.sum(e, axis=0, keepdims=True)
    acc = jax.lax.dot_general(
        c_ref[...], gate, (((1,), (0,)), ((), ())),
        preferred_element_type=jnp.float32)
    out_ref[...] = acc.reshape(1, NUM_BLOCKS, BLOCK_SIZE, BLOCK_SIZE,
                               acc.shape[-1])


def kernel(tensor, W1, b1, W2, b2, cluster_logits):
    B, SEQ, _ = tensor.shape
    b1r = b1.reshape(1, HIDDEN_SIZE)
    b2r = b2.reshape(NUM_CLUSTERS, 1)
    cT = cluster_logits.reshape(NUM_CLUSTERS, -1).T

    grid = (B, SEQ // S_TILE)
    out = pl.pallas_call(
        _fused_kernel,
        grid=grid,
        in_specs=[
            pl.BlockSpec((1, S_TILE, HIDDEN_DIM), lambda b, s: (b, s, 0)),
            pl.BlockSpec((HIDDEN_DIM, HIDDEN_SIZE), lambda b, s: (0, 0)),
            pl.BlockSpec((1, HIDDEN_SIZE), lambda b, s: (0, 0)),
            pl.BlockSpec((HIDDEN_SIZE, NUM_CLUSTERS), lambda b, s: (0, 0)),
            pl.BlockSpec((NUM_CLUSTERS, 1), lambda b, s: (0, 0)),
            pl.BlockSpec((NUM_BLOCKS * BLOCK_SIZE * BLOCK_SIZE, NUM_CLUSTERS),
                         lambda b, s: (0, 0)),
        ],
        out_specs=pl.BlockSpec(
            (1, NUM_BLOCKS, BLOCK_SIZE, BLOCK_SIZE, S_TILE),
            lambda b, s: (b, 0, 0, 0, s)),
        out_shape=jax.ShapeDtypeStruct(
            (B, NUM_BLOCKS, BLOCK_SIZE, BLOCK_SIZE, SEQ), jnp.float32),
    )(tensor, W1, b1r, W2, b2r, cT)
    return jnp.transpose(out, (0, 4, 1, 2, 3))


# speedup vs baseline: 1.0644x; 1.0008x over previous
import jax
import jax.numpy as jnp
from jax.experimental import pallas as pl

HIDDEN_DIM = 1024
NUM_BLOCKS = 32
BLOCK_SIZE = 16
NUM_CLUSTERS = 4
HIDDEN_SIZE = 128

S_TILE = 512


def _fused_kernel(x_ref, w1_ref, b1_ref, w2_ref, b2_ref, c_ref, out_ref):
    x = x_ref[0]
    h = jax.lax.dot_general(
        x.astype(jnp.bfloat16), w1_ref[...].astype(jnp.bfloat16),
        (((1,), (0,)), ((), ())),
        preferred_element_type=jnp.float32) + b1_ref[...]
    h = 0.5 * h * (1.0 + jax.lax.erf(h * 0.7071067811865476))
    g = jax.lax.dot_general(
        w2_ref[...], h, (((0,), (1,)), ((), ())),
        preferred_element_type=jnp.float32) + b2_ref[...]
    g = g - jnp.max(g, axis=0, keepdims=True)
    e = jnp.exp(g)
    gate = e / jnp.sum(e, axis=0, keepdims=True)
    acc = jax.lax.dot_general(
        c_ref[...], gate, (((1,), (0,)), ((), ())),
        preferred_element_type=jnp.float32)
    out_ref[...] = acc.reshape(1, NUM_BLOCKS, BLOCK_SIZE, BLOCK_SIZE,
                               acc.shape[-1])


def kernel(tensor, W1, b1, W2, b2, cluster_logits):
    B, SEQ, _ = tensor.shape
    b1r = b1.reshape(1, HIDDEN_SIZE)
    b2r = b2.reshape(NUM_CLUSTERS, 1)
    cT = cluster_logits.reshape(NUM_CLUSTERS, -1).T

    grid = (B, SEQ // S_TILE)
    out = pl.pallas_call(
        _fused_kernel,
        grid=grid,
        in_specs=[
            pl.BlockSpec((1, S_TILE, HIDDEN_DIM), lambda b, s: (b, s, 0)),
            pl.BlockSpec((HIDDEN_DIM, HIDDEN_SIZE), lambda b, s: (0, 0)),
            pl.BlockSpec((1, HIDDEN_SIZE), lambda b, s: (0, 0)),
            pl.BlockSpec((HIDDEN_SIZE, NUM_CLUSTERS), lambda b, s: (0, 0)),
            pl.BlockSpec((NUM_CLUSTERS, 1), lambda b, s: (0, 0)),
            pl.BlockSpec((NUM_BLOCKS * BLOCK_SIZE * BLOCK_SIZE, NUM_CLUSTERS),
                         lambda b, s: (0, 0)),
        ],
        out_specs=pl.BlockSpec(
            (1, NUM_BLOCKS, BLOCK_SIZE, BLOCK_SIZE, S_TILE),
            lambda b, s: (b, 0, 0, 0, s)),
        out_shape=jax.ShapeDtypeStruct(
            (B, NUM_BLOCKS, BLOCK_SIZE, BLOCK_SIZE, SEQ), jnp.float32),
    )(tensor, W1, b1r, W2, b2r, cT)
    return jnp.transpose(out, (0, 4, 1, 2, 3))


# speedup vs baseline: 1.0651x; 1.0007x over previous
import jax
import jax.numpy as jnp
from jax.experimental import pallas as pl

HIDDEN_DIM = 1024
NUM_BLOCKS = 32
BLOCK_SIZE = 16
NUM_CLUSTERS = 4
HIDDEN_SIZE = 128

S_TILE = 512


def _fused_kernel(x_ref, w1_ref, b1_ref, w2_ref, b2_ref, c_ref, out_ref):
    x = x_ref[0]
    h = x @ w1_ref[...] + b1_ref[...]
    h = 0.5 * h * (1.0 + jax.lax.erf(h * 0.7071067811865476))
    g = jax.lax.dot_general(
        w2_ref[...], h, (((0,), (1,)), ((), ())),
        preferred_element_type=jnp.float32) + b2_ref[...]
    g = g - jnp.max(g, axis=0, keepdims=True)
    e = jnp.exp(g)
    gate = e / jnp.sum(e, axis=0, keepdims=True)
    acc = jax.lax.dot_general(
        c_ref[...], gate, (((1,), (0,)), ((), ())),
        preferred_element_type=jnp.float32)
    out_ref[...] = acc.reshape(1, NUM_BLOCKS, BLOCK_SIZE, BLOCK_SIZE,
                               acc.shape[-1])


def kernel(tensor, W1, b1, W2, b2, cluster_logits):
    B, SEQ, _ = tensor.shape
    b1r = b1.reshape(1, HIDDEN_SIZE)
    b2r = b2.reshape(NUM_CLUSTERS, 1)
    cT = cluster_logits.reshape(NUM_CLUSTERS, -1).T

    grid = (B, SEQ // S_TILE)
    out = pl.pallas_call(
        _fused_kernel,
        grid=grid,
        in_specs=[
            pl.BlockSpec((1, S_TILE, HIDDEN_DIM), lambda b, s: (b, s, 0)),
            pl.BlockSpec((HIDDEN_DIM, HIDDEN_SIZE), lambda b, s: (0, 0)),
            pl.BlockSpec((1, HIDDEN_SIZE), lambda b, s: (0, 0)),
            pl.BlockSpec((HIDDEN_SIZE, NUM_CLUSTERS), lambda b, s: (0, 0)),
            pl.BlockSpec((NUM_CLUSTERS, 1), lambda b, s: (0, 0)),
            pl.BlockSpec((NUM_BLOCKS * BLOCK_SIZE * BLOCK_SIZE, NUM_CLUSTERS),
                         lambda b, s: (0, 0)),
        ],
        out_specs=pl.BlockSpec(
            (1, NUM_BLOCKS, BLOCK_SIZE, BLOCK_SIZE, S_TILE),
            lambda b, s: (b, 0, 0, 0, s)),
        out_shape=jax.ShapeDtypeStruct(
            (B, NUM_BLOCKS, BLOCK_SIZE, BLOCK_SIZE, SEQ), jnp.float32),
    )(tensor, W1, b1r, W2, b2r, cT)
    return jnp.transpose(out, (0, 4, 1, 2, 3))
